# baseline (speedup 1.0000x reference)
import jax
import jax.numpy as jnp
from jax import lax
from jax.experimental import pallas as pl
from jax.experimental.pallas import tpu as pltpu
from jax.experimental.pallas import tpu_sc as plsc

_CONF_DELTA = 0.05
_NUM_OBJ_CLASSES = 42

_N_ROWS = 262144
_N_FEATS = 204

_NC = 2
_NS = 16
_NW = _NC * _NS

_BATCH_PER_W = _N_ROWS // _NW
_W = 1024
_NCHUNKS = _BATCH_PER_W // _W
_LANES = 16
_SEGS = _W // _LANES


def _activation_cols(D):
    num_obj_feats = _NUM_OBJ_CLASSES - 2
    num_obj_points = num_obj_feats * 2
    obj_acts_idx = num_obj_points + 1 + num_obj_points + 2 + 1
    return [0, num_obj_points + 1] + list(range(obj_acts_idx, D))


_ROWS = _activation_cols(_N_FEATS)

_NFB = (_N_FEATS + 7) // 8
_TOUCH = {}
for _r in _ROWS:
    _TOUCH.setdefault(_r // 8, []).append(_r % 8)
_TFB = sorted(_TOUCH)
_UFB = [fb for fb in range(_NFB) if fb not in _TOUCH]
_NBUF = len(_TFB)
_FBH = {fb: min(8, _N_FEATS - fb * 8) for fb in range(_NFB)}


def _sc_body(feat_hbm, dvec_hbm, out_hbm, bufs, dvec_v,
             sems_in, sems_out, sem_hh):
    wid = lax.axis_index("c") * _NS + lax.axis_index("s")
    col0 = wid * _BATCH_PER_W

    pltpu.sync_copy(dvec_hbm, dvec_v)
    dv = dvec_v[...]

    def fb_slice(ref, fb, chunk):
        return ref.at[pl.ds(fb * 8, _FBH[fb]),
                      pl.ds(col0 + chunk * _W, _W)]

    def buf_ref(b, fb):
        h = _FBH[fb]
        return bufs[b] if h == 8 else bufs[b].at[pl.ds(0, h), :]

    def start_in(b, fb, chunk):
        pltpu.async_copy(fb_slice(feat_hbm, fb, chunk), buf_ref(b, fb),
                         sems_in[b])

    def wait_in(b, fb):
        pltpu.make_async_copy(fb_slice(feat_hbm, fb, 0), buf_ref(b, fb),
                              sems_in[b]).wait()

    def start_out(b, fb, chunk):
        pltpu.async_copy(buf_ref(b, fb), fb_slice(out_hbm, fb, chunk),
                         sems_out[b])

    def wait_out(b, fb):
        pltpu.make_async_copy(buf_ref(b, fb), fb_slice(out_hbm, fb, 0),
                              sems_out[b]).wait()

    def compute(b, fb):
        buf = bufs[b]
        rows = _TOUCH[fb]

        @plsc.parallel_loop(0, _SEGS, 1, unroll=1)
        def seg_body(i):
            c = i * _LANES
            for r in rows:
                v = buf[r, pl.ds(c, _LANES)]
                t = jnp.minimum(jnp.maximum(v + dv, 0.0), 1.0)
                buf[r, pl.ds(c, _LANES)] = jnp.where(v == 0.0, v, t)

    for idx, fb in enumerate(_TFB):
        start_in(idx, fb, 0)

    def chunk_body(g, carry):
        for idx, fb in enumerate(_TFB):
            b = idx % _NBUF
            wait_in(b, fb)
            compute(b, fb)
            start_out(b, fb, g)
            idx2 = idx + _NBUF - 2
            b2 = idx2 % _NBUF
            fb2 = _TFB[idx2 % _NBUF]
            g2 = g + idx2 // _NBUF
            fb3 = _TFB[(idx2 - _NBUF) % _NBUF]

            @pl.when(jnp.logical_and(g2 >= 1, g2 < _NCHUNKS))
            def _():
                wait_out(b2, fb3)
                start_in(b2, fb2, g2)

        for fb in _UFB:
            pltpu.async_copy(fb_slice(feat_hbm, fb, g),
                             fb_slice(out_hbm, fb, g), sem_hh)

        @pl.when(g >= 1)
        def _():
            for fb in _UFB:
                pltpu.make_async_copy(fb_slice(feat_hbm, fb, 0),
                                      fb_slice(out_hbm, fb, 0),
                                      sem_hh).wait()

        return carry

    lax.fori_loop(0, _NCHUNKS, chunk_body, 0)

    for fb in _UFB:
        pltpu.make_async_copy(fb_slice(feat_hbm, fb, 0),
                              fb_slice(out_hbm, fb, 0), sem_hh).wait()
    for idx, fb in enumerate(_TFB):
        wait_out(idx, fb)


@jax.jit
def kernel(features):
    delta = jax.random.uniform(
        jax.random.key(1), (), dtype=jnp.float32,
        minval=-_CONF_DELTA, maxval=_CONF_DELTA,
    )
    dvec = jnp.full((_LANES,), delta, jnp.float32)

    mesh = plsc.VectorSubcoreMesh(
        core_axis_name="c", subcore_axis_name="s",
        num_cores=_NC, num_subcores=_NS)

    run = pl.kernel(
        _sc_body,
        out_type=jax.ShapeDtypeStruct((_N_FEATS, _N_ROWS), jnp.float32),
        mesh=mesh,
        compiler_params=pltpu.CompilerParams(needs_layout_passes=False),
        scratch_types=dict(
            bufs=[pltpu.VMEM((8, _W), jnp.float32) for _ in range(_NBUF)],
            dvec_v=pltpu.VMEM((_LANES,), jnp.float32),
            sems_in=[pltpu.SemaphoreType.DMA for _ in range(_NBUF)],
            sems_out=[pltpu.SemaphoreType.DMA for _ in range(_NBUF)],
            sem_hh=pltpu.SemaphoreType.DMA,
        ),
    )
    out_t = run(features.T, dvec)
    return out_t.T


# speedup vs baseline: 27.3827x; 27.3827x over previous
import jax
import jax.numpy as jnp
from jax import lax
from jax.experimental import pallas as pl
from jax.experimental.pallas import tpu as pltpu
from jax.experimental.pallas import tpu_sc as plsc

_CONF_DELTA = 0.05
_NUM_OBJ_CLASSES = 42

_N_ROWS = 262144
_N_FEATS = 204

_NC = 2
_NS = 16
_NW = _NC * _NS

_BATCH_PER_W = _N_ROWS // _NW
_W = 1024
_NCHUNKS = _BATCH_PER_W // _W
_LANES = 16
_SEGS = _W // _LANES
_NBUF = 13


def _activation_cols(D):
    num_obj_feats = _NUM_OBJ_CLASSES - 2
    num_obj_points = num_obj_feats * 2
    obj_acts_idx = num_obj_points + 1 + num_obj_points + 2 + 1
    return [0, num_obj_points + 1] + list(range(obj_acts_idx, D))


_ROWS = _activation_cols(_N_FEATS)

_NFB = (_N_FEATS + 7) // 8
_TOUCH = {}
for _r in _ROWS:
    _TOUCH.setdefault(_r // 8, []).append(_r % 8)
_FBH = {fb: min(8, _N_FEATS - fb * 8) for fb in range(_NFB)}
_UNITS = _NCHUNKS * _NFB


def _sc_body(feat_hbm, dvec_hbm, out_hbm, bufs, dvec_v, sems_in, sems_out):
    wid = lax.axis_index("c") * _NS + lax.axis_index("s")
    col0 = wid * _BATCH_PER_W

    pltpu.sync_copy(dvec_hbm, dvec_v)
    dv = dvec_v[...]

    def fb_slice(ref, fb, chunk):
        return ref.at[pl.ds(fb * 8, _FBH[fb]),
                      pl.ds(col0 + chunk * _W, _W)]

    def buf_ref(b, fb):
        h = _FBH[fb]
        return bufs[b] if h == 8 else bufs[b].at[pl.ds(0, h), :]

    def start_in(b, fb, chunk):
        pltpu.async_copy(fb_slice(feat_hbm, fb, chunk), buf_ref(b, fb),
                         sems_in[b])

    def wait_in(b, fb):
        pltpu.make_async_copy(fb_slice(feat_hbm, fb, 0), buf_ref(b, fb),
                              sems_in[b]).wait()

    def start_out(b, fb, chunk):
        pltpu.async_copy(buf_ref(b, fb), fb_slice(out_hbm, fb, chunk),
                         sems_out[b])

    def wait_out(b, fb):
        pltpu.make_async_copy(buf_ref(b, fb), fb_slice(out_hbm, fb, 0),
                              sems_out[b]).wait()

    def compute(b, fb):
        buf = bufs[b]
        rows = _TOUCH[fb]

        @plsc.parallel_loop(0, _SEGS, 1, unroll=1)
        def seg_body(i):
            c = i * _LANES
            for r in rows:
                v = buf[r, pl.ds(c, _LANES)]
                t = jnp.minimum(jnp.maximum(v + dv, 0.0), 1.0)
                buf[r, pl.ds(c, _LANES)] = jnp.where(v == 0.0, v, t)

    for fb in range(_NBUF):
        start_in(fb, fb, 0)

    def chunk_body(g, carry):
        for fb in range(_NFB):
            b = fb % _NBUF
            wait_in(b, fb)
            if fb in _TOUCH:
                compute(b, fb)
            start_out(b, fb, g)
            u2 = g * _NFB + fb + (_NBUF - 2)
            fb2 = (fb + _NBUF - 2) % _NFB
            b2 = fb2 % _NBUF
            fb3 = (fb2 + _NBUF) % _NFB
            g2 = g + (fb + _NBUF - 2) // _NFB

            @pl.when(jnp.logical_and(u2 >= _NBUF, u2 < _UNITS))
            def _():
                wait_out(b2, fb3)
                start_in(b2, fb2, g2)

        return carry

    lax.fori_loop(0, _NCHUNKS, chunk_body, 0)

    for j in range(_NBUF):
        wait_out(j, _NBUF + j)


@jax.jit
def kernel(features):
    delta = jax.random.uniform(
        jax.random.key(1), (), dtype=jnp.float32,
        minval=-_CONF_DELTA, maxval=_CONF_DELTA,
    )
    dvec = jnp.full((_LANES,), delta, jnp.float32)

    mesh = plsc.VectorSubcoreMesh(
        core_axis_name="c", subcore_axis_name="s",
        num_cores=_NC, num_subcores=_NS)

    run = pl.kernel(
        _sc_body,
        out_type=jax.ShapeDtypeStruct((_N_FEATS, _N_ROWS), jnp.float32),
        mesh=mesh,
        compiler_params=pltpu.CompilerParams(needs_layout_passes=False),
        scratch_types=dict(
            bufs=[pltpu.VMEM((8, _W), jnp.float32) for _ in range(_NBUF)],
            dvec_v=pltpu.VMEM((_LANES,), jnp.float32),
            sems_in=[pltpu.SemaphoreType.DMA for _ in range(_NBUF)],
            sems_out=[pltpu.SemaphoreType.DMA for _ in range(_NBUF)],
        ),
    )
    out_t = run(features.T, dvec)
    return out_t.T
